# baseline (speedup 1.0000x reference)
import functools

import jax
import jax.numpy as jnp
from jax import lax
from jax.experimental import pallas as pl
from jax.experimental.pallas import tpu as pltpu
from jax.experimental.pallas import tpu_sc as plsc

_NC = 2
_NS = 16
_NW = _NC * _NS
_L = 16


def _sc_aggregate(h, pk, zinit, n_chunks, chunk):
    N, D = h.shape
    rpt = (N // _NS) // 8 * 8
    tail = N - rpt * _NS

    mesh = plsc.VectorSubcoreMesh(core_axis_name="c", subcore_axis_name="s",
                                  num_cores=_NC, num_subcores=_NS)

    def body(h_hbm, pk_hbm, zin_hbm, out_hbm, acc_sh,
             pkt0, pkt1, pkt2, rows0, rows1, rows2,
             semg0, semg1, semg2, sems0, sems1, sems2, semi0, semi1, semi2):
        cid = lax.axis_index("c")
        sid = lax.axis_index("s")
        wid = cid * _NS + sid
        pkts = (pkt0, pkt1, pkt2)
        rows = (rows0, rows1, rows2)
        semg = (semg0, semg1, semg2)
        sems = (sems0, sems1, sems2)
        semi = (semi0, semi1, semi2)

        pltpu.sync_copy(zin_hbm.at[pl.ds(0, rpt)], acc_sh.at[pl.ds(sid * rpt, rpt)])
        @pl.when(sid == 0)
        def _init_tail():
            pltpu.sync_copy(zin_hbm.at[pl.ds(0, tail)],
                            acc_sh.at[pl.ds(rpt * _NS, tail)])
        plsc.subcore_barrier()

        def issue_idx(q, b):
            pltpu.async_copy(pk_hbm.at[wid, q], pkts[b], semi[b])

        def wait_idx(b):
            pltpu.make_async_copy(pk_hbm.at[wid, 0], pkts[b], semi[b]).wait()

        def issue_gather(b):
            pltpu.async_copy(h_hbm.at[pkts[b].at[0]], rows[b], semg[b])

        def wait_gather(b):
            pltpu.make_async_copy(h_hbm.at[pkts[b].at[0]], rows[b], semg[b]).wait()

        def issue_scatter(b):
            pltpu.async_copy(rows[b], acc_sh.at[pkts[b].at[1]], sems[b], add=True)

        def wait_scatter(b):
            pltpu.make_async_copy(rows[b], acc_sh.at[pkts[b].at[1]], sems[b]).wait()

        def multiply(b):
            rv, pkv = rows[b], pkts[b]

            @plsc.parallel_loop(0, chunk // _L)
            def _mul(g):
                mvec = plsc.bitcast(pkv[2, pl.ds(g * _L, _L)], jnp.float32)
                for l in range(_L):
                    e = g * _L + l
                    mv = mvec[l]
                    for j in range(D // _L):
                        sl = (e, pl.ds(j * _L, _L))
                        rv[sl] = rv[sl] * mv

        def phase(q, b):
            bz = (b + 2) % 3
            by = (b + 1) % 3
            @pl.when(q + 1 < n_chunks)
            def _():
                wait_idx(by)
                issue_gather(by)
            wait_gather(b)
            multiply(b)
            issue_scatter(b)
            @pl.when(q > 0)
            def _():
                wait_scatter(bz)
            @pl.when(q + 2 < n_chunks)
            def _():
                issue_idx(q + 2, bz)

        issue_idx(0, 0)
        issue_idx(1, 1)
        wait_idx(0)
        issue_gather(0)

        def loop_body(s, carry):
            q0 = s * 3
            for t in range(3):
                @pl.when(q0 + t < n_chunks)
                def _(t=t):
                    phase(q0 + t, t)
            return carry

        lax.fori_loop(0, (n_chunks + 2) // 3, loop_body, 0)
        wait_scatter((n_chunks - 1) % 3)
        plsc.subcore_barrier()
        pltpu.sync_copy(acc_sh.at[pl.ds(sid * rpt, rpt)],
                        out_hbm.at[cid, pl.ds(sid * rpt, rpt)])
        @pl.when(sid == 0)
        def _drain_tail():
            pltpu.sync_copy(acc_sh.at[pl.ds(rpt * _NS, tail)],
                            out_hbm.at[cid, pl.ds(rpt * _NS, tail)])

    run = pl.kernel(
        body,
        out_type=jax.ShapeDtypeStruct((_NC, N, D), jnp.float32),
        mesh=mesh,
        scratch_types=[
            pltpu.VMEM_SHARED((N, D), jnp.float32),
        ] + [pltpu.VMEM((3, chunk), jnp.int32)] * 3
          + [pltpu.VMEM((chunk, D), jnp.float32)] * 3
          + [pltpu.SemaphoreType.DMA] * 9,
        compiler_params=pltpu.CompilerParams(needs_layout_passes=False),
    )
    return run(h, pk, zinit)


def _tc_mlp(h, partials, W1, b1, W2, b2, a):
    N, D = h.shape
    R = 1000

    def body(h_ref, p_ref, w1_ref, b1_ref, w2_ref, b2_ref, a_ref, o_ref):
        h2 = h_ref[...] + p_ref[0] + p_ref[1]
        z = jnp.dot(h2, w1_ref[...], preferred_element_type=jnp.float32)
        z = jnp.maximum(z + b1_ref[...], 0.0)
        z = jnp.dot(z, w2_ref[...], preferred_element_type=jnp.float32)
        z = z + b2_ref[...]
        av = a_ref[0, 0]
        o_ref[...] = jnp.where(z >= 0, z, av * z)

    return pl.pallas_call(
        body,
        grid=(N // R,),
        in_specs=[
            pl.BlockSpec((R, D), lambda i: (i, 0)),
            pl.BlockSpec((_NC, R, D), lambda i: (0, i, 0)),
            pl.BlockSpec((D, D), lambda i: (0, 0)),
            pl.BlockSpec((1, D), lambda i: (0, 0)),
            pl.BlockSpec((D, D), lambda i: (0, 0)),
            pl.BlockSpec((1, D), lambda i: (0, 0)),
            pl.BlockSpec((1, 1), lambda i: (0, 0)),
        ],
        out_specs=pl.BlockSpec((R, D), lambda i: (i, 0)),
        out_shape=jax.ShapeDtypeStruct((N, D), jnp.float32),
    )(h, partials, W1, b1.reshape(1, D), W2, b2.reshape(1, D),
      a.reshape(1, 1))


def kernel(h, edge_index, edge_mask, snorm_n, W1, b1, W2, b2, prelu_a):
    del snorm_n
    N, D = h.shape
    E = edge_index.shape[1]
    eper = E // _NW
    chunk = 80
    n_chunks = eper // chunk
    srcr = edge_index[0].reshape(_NW, n_chunks, 1, chunk)
    dstr = edge_index[1].reshape(_NW, n_chunks, 1, chunk)
    maskr = lax.bitcast_convert_type(edge_mask, jnp.int32).reshape(
        _NW, n_chunks, 1, chunk)
    pk = jnp.concatenate([srcr, dstr, maskr], axis=2)
    zinit = jnp.zeros((N // _NS // 8 * 8, D), jnp.float32)
    partials = _sc_aggregate(h, pk, zinit, n_chunks, chunk)
    return _tc_mlp(h, partials, W1, b1, W2, b2, prelu_a)


# speedup vs baseline: 1.0973x; 1.0973x over previous
import functools

import jax
import jax.numpy as jnp
from jax import lax
from jax.experimental import pallas as pl
from jax.experimental.pallas import tpu as pltpu
from jax.experimental.pallas import tpu_sc as plsc

_NC = 2
_NS = 16
_NW = _NC * _NS
_L = 16


def _sc_aggregate(h, pk, zinit, n_chunks, chunk):
    N, D = h.shape
    rpt = (N // _NS) // 8 * 8
    tail = N - rpt * _NS

    mesh = plsc.VectorSubcoreMesh(core_axis_name="c", subcore_axis_name="s",
                                  num_cores=_NC, num_subcores=_NS)

    def body(h_hbm, pk_hbm, zin_hbm, out_hbm, acc_sh,
             pkt0, pkt1, pkt2, rows0, rows1, rows2,
             semg0, semg1, semg2, sems0, sems1, sems2, semi0, semi1, semi2):
        cid = lax.axis_index("c")
        sid = lax.axis_index("s")
        wid = cid * _NS + sid
        pkts = (pkt0, pkt1, pkt2)
        rows = (rows0, rows1, rows2)
        semg = (semg0, semg1, semg2)
        sems = (sems0, sems1, sems2)
        semi = (semi0, semi1, semi2)

        pltpu.sync_copy(zin_hbm.at[pl.ds(0, rpt)], acc_sh.at[pl.ds(sid * rpt, rpt)])
        @pl.when(sid == 0)
        def _init_tail():
            pltpu.sync_copy(zin_hbm.at[pl.ds(0, tail)],
                            acc_sh.at[pl.ds(rpt * _NS, tail)])
        plsc.subcore_barrier()

        def issue_idx(q, b):
            pltpu.async_copy(pk_hbm.at[wid, q], pkts[b], semi[b])

        def wait_idx(b):
            pltpu.make_async_copy(pk_hbm.at[wid, 0], pkts[b], semi[b]).wait()

        def issue_gather(b):
            pltpu.async_copy(h_hbm.at[pkts[b].at[0]], rows[b], semg[b])

        def wait_gather(b):
            pltpu.make_async_copy(h_hbm.at[pkts[b].at[0]], rows[b], semg[b]).wait()

        def issue_scatter(b):
            pltpu.async_copy(rows[b], acc_sh.at[pkts[b].at[1]], sems[b], add=True)

        def wait_scatter(b):
            pltpu.make_async_copy(rows[b], acc_sh.at[pkts[b].at[1]], sems[b]).wait()

        def multiply(b):
            rv, pkv = rows[b], pkts[b]
            two = jnp.full((_L,), 2, jnp.int32)

            @plsc.parallel_loop(0, chunk, unroll=16)
            def _mul(e):
                mvi = plsc.load_gather(pkv, [two, jnp.full((_L,), e, jnp.int32)])
                mv = plsc.bitcast(mvi, jnp.float32)
                for j in range(D // _L):
                    sl = (e, pl.ds(j * _L, _L))
                    rv[sl] = rv[sl] * mv

        def phase(q, b):
            bz = (b + 2) % 3
            by = (b + 1) % 3
            @pl.when(q + 1 < n_chunks)
            def _():
                wait_idx(by)
                issue_gather(by)
            wait_gather(b)
            multiply(b)
            issue_scatter(b)
            @pl.when(q > 0)
            def _():
                wait_scatter(bz)
            @pl.when(q + 2 < n_chunks)
            def _():
                issue_idx(q + 2, bz)

        issue_idx(0, 0)
        issue_idx(1, 1)
        wait_idx(0)
        issue_gather(0)

        def loop_body(s, carry):
            q0 = s * 3
            for t in range(3):
                @pl.when(q0 + t < n_chunks)
                def _(t=t):
                    phase(q0 + t, t)
            return carry

        lax.fori_loop(0, (n_chunks + 2) // 3, loop_body, 0)
        wait_scatter((n_chunks - 1) % 3)
        plsc.subcore_barrier()
        pltpu.sync_copy(acc_sh.at[pl.ds(sid * rpt, rpt)],
                        out_hbm.at[cid, pl.ds(sid * rpt, rpt)])
        @pl.when(sid == 0)
        def _drain_tail():
            pltpu.sync_copy(acc_sh.at[pl.ds(rpt * _NS, tail)],
                            out_hbm.at[cid, pl.ds(rpt * _NS, tail)])

    run = pl.kernel(
        body,
        out_type=jax.ShapeDtypeStruct((_NC, N, D), jnp.float32),
        mesh=mesh,
        scratch_types=[
            pltpu.VMEM_SHARED((N, D), jnp.float32),
        ] + [pltpu.VMEM((3, chunk), jnp.int32)] * 3
          + [pltpu.VMEM((chunk, D), jnp.float32)] * 3
          + [pltpu.SemaphoreType.DMA] * 9,
        compiler_params=pltpu.CompilerParams(needs_layout_passes=False),
    )
    return run(h, pk, zinit)


def _tc_mlp(h, partials, W1, b1, W2, b2, a):
    N, D = h.shape
    R = 1000

    def body(h_ref, p_ref, w1_ref, b1_ref, w2_ref, b2_ref, a_ref, o_ref):
        h2 = h_ref[...] + p_ref[0] + p_ref[1]
        z = jnp.dot(h2, w1_ref[...], preferred_element_type=jnp.float32)
        z = jnp.maximum(z + b1_ref[...], 0.0)
        z = jnp.dot(z, w2_ref[...], preferred_element_type=jnp.float32)
        z = z + b2_ref[...]
        av = a_ref[0, 0]
        o_ref[...] = jnp.where(z >= 0, z, av * z)

    return pl.pallas_call(
        body,
        grid=(N // R,),
        in_specs=[
            pl.BlockSpec((R, D), lambda i: (i, 0)),
            pl.BlockSpec((_NC, R, D), lambda i: (0, i, 0)),
            pl.BlockSpec((D, D), lambda i: (0, 0)),
            pl.BlockSpec((1, D), lambda i: (0, 0)),
            pl.BlockSpec((D, D), lambda i: (0, 0)),
            pl.BlockSpec((1, D), lambda i: (0, 0)),
            pl.BlockSpec((1, 1), lambda i: (0, 0)),
        ],
        out_specs=pl.BlockSpec((R, D), lambda i: (i, 0)),
        out_shape=jax.ShapeDtypeStruct((N, D), jnp.float32),
    )(h, partials, W1, b1.reshape(1, D), W2, b2.reshape(1, D),
      a.reshape(1, 1))


def kernel(h, edge_index, edge_mask, snorm_n, W1, b1, W2, b2, prelu_a):
    del snorm_n
    N, D = h.shape
    E = edge_index.shape[1]
    eper = E // _NW
    chunk = 80
    n_chunks = eper // chunk
    srcr = edge_index[0].reshape(_NW, n_chunks, 1, chunk)
    dstr = edge_index[1].reshape(_NW, n_chunks, 1, chunk)
    maskr = lax.bitcast_convert_type(edge_mask, jnp.int32).reshape(
        _NW, n_chunks, 1, chunk)
    pk = jnp.concatenate([srcr, dstr, maskr], axis=2)
    zinit = jnp.zeros((N // _NS // 8 * 8, D), jnp.float32)
    partials = _sc_aggregate(h, pk, zinit, n_chunks, chunk)
    return _tc_mlp(h, partials, W1, b1, W2, b2, prelu_a)


# speedup vs baseline: 1.1181x; 1.0190x over previous
import functools

import jax
import jax.numpy as jnp
from jax import lax
from jax.experimental import pallas as pl
from jax.experimental.pallas import tpu as pltpu
from jax.experimental.pallas import tpu_sc as plsc

_NC = 2
_NS = 16
_NW = _NC * _NS
_L = 16


def _sc_aggregate(h, pk, zinit, n_chunks, chunk):
    N, D = h.shape
    rpt = (N // _NS) // 8 * 8
    tail = N - rpt * _NS

    mesh = plsc.VectorSubcoreMesh(core_axis_name="c", subcore_axis_name="s",
                                  num_cores=_NC, num_subcores=_NS)

    def body(h_hbm, pk_hbm, zin_hbm, out_hbm, acc_sh,
             pkt0, pkt1, pkt2, pkt3, pkt4, pkt5,
             rows0, rows1, rows2, rows3,
             semg0, semg1, semg2, semg3, sems0, sems1, sems2, sems3,
             semi0, semi1, semi2, semi3, semi4, semi5):
        cid = lax.axis_index("c")
        sid = lax.axis_index("s")
        wid = cid * _NS + sid
        pkts = (pkt0, pkt1, pkt2, pkt3, pkt4, pkt5)
        rows = (rows0, rows1, rows2, rows3)
        semg = (semg0, semg1, semg2, semg3)
        sems = (sems0, sems1, sems2, sems3)
        semi = (semi0, semi1, semi2, semi3, semi4, semi5)

        pltpu.sync_copy(zin_hbm.at[pl.ds(0, rpt)], acc_sh.at[pl.ds(sid * rpt, rpt)])
        @pl.when(sid == 0)
        def _init_tail():
            pltpu.sync_copy(zin_hbm.at[pl.ds(0, tail)],
                            acc_sh.at[pl.ds(rpt * _NS, tail)])
        plsc.subcore_barrier()

        def issue_idx(q, bi):
            pltpu.async_copy(pk_hbm.at[wid, q], pkts[bi], semi[bi])

        def wait_idx(bi):
            pltpu.make_async_copy(pk_hbm.at[wid, 0], pkts[bi], semi[bi]).wait()

        def issue_gather(br, bi):
            pltpu.async_copy(h_hbm.at[pkts[bi].at[0]], rows[br], semg[br])

        def wait_gather(br, bi):
            pltpu.make_async_copy(h_hbm.at[pkts[bi].at[0]], rows[br],
                                  semg[br]).wait()

        def issue_scatter(br, bi):
            pltpu.async_copy(rows[br], acc_sh.at[pkts[bi].at[1]], sems[br],
                             add=True)

        def wait_scatter(br, bi):
            pltpu.make_async_copy(rows[br], acc_sh.at[pkts[bi].at[1]],
                                  sems[br]).wait()

        def multiply(br, bi):
            rv, pkv = rows[br], pkts[bi]
            two = jnp.full((_L,), 2, jnp.int32)

            @plsc.parallel_loop(0, chunk, unroll=16)
            def _mul(e):
                mvi = plsc.load_gather(pkv, [two, jnp.full((_L,), e, jnp.int32)])
                mv = plsc.bitcast(mvi, jnp.float32)
                for j in range(D // _L):
                    sl = (e, pl.ds(j * _L, _L))
                    rv[sl] = rv[sl] * mv

        def phase(q, t):
            br = t % 4
            br2 = (t + 2) % 4
            bi = t % 6
            bi2 = (t + 2) % 6
            bi3 = (t + 3) % 6
            @pl.when(q >= 2)
            def _():
                wait_scatter(br2, (t + 4) % 6)
            @pl.when(q + 2 < n_chunks)
            def _():
                wait_idx(bi2)
                issue_gather(br2, bi2)
            wait_gather(br, bi)
            multiply(br, bi)
            issue_scatter(br, bi)
            @pl.when(q + 3 < n_chunks)
            def _():
                issue_idx(q + 3, bi3)

        issue_idx(0, 0)
        issue_idx(1, 1)
        issue_idx(2, 2)
        wait_idx(0)
        issue_gather(0, 0)
        wait_idx(1)
        issue_gather(1, 1)

        def loop_body(s, carry):
            q0 = s * 12
            for t in range(12):
                @pl.when(q0 + t < n_chunks)
                def _(t=t):
                    phase(q0 + t, t)
            return carry

        lax.fori_loop(0, (n_chunks + 11) // 12, loop_body, 0)
        wait_scatter((n_chunks - 2) % 4, (n_chunks - 2) % 6)
        wait_scatter((n_chunks - 1) % 4, (n_chunks - 1) % 6)
        plsc.subcore_barrier()
        pltpu.sync_copy(acc_sh.at[pl.ds(sid * rpt, rpt)],
                        out_hbm.at[cid, pl.ds(sid * rpt, rpt)])
        @pl.when(sid == 0)
        def _drain_tail():
            pltpu.sync_copy(acc_sh.at[pl.ds(rpt * _NS, tail)],
                            out_hbm.at[cid, pl.ds(rpt * _NS, tail)])

    run = pl.kernel(
        body,
        out_type=jax.ShapeDtypeStruct((_NC, N, D), jnp.float32),
        mesh=mesh,
        scratch_types=[
            pltpu.VMEM_SHARED((N, D), jnp.float32),
        ] + [pltpu.VMEM((3, chunk), jnp.int32)] * 6
          + [pltpu.VMEM((chunk, D), jnp.float32)] * 4
          + [pltpu.SemaphoreType.DMA] * 14,
        compiler_params=pltpu.CompilerParams(needs_layout_passes=False),
    )
    return run(h, pk, zinit)


def _tc_mlp(h, partials, W1, b1, W2, b2, a):
    N, D = h.shape
    R = 1000

    def body(h_ref, p_ref, w1_ref, b1_ref, w2_ref, b2_ref, a_ref, o_ref):
        h2 = h_ref[...] + p_ref[0] + p_ref[1]
        z = jnp.dot(h2, w1_ref[...], preferred_element_type=jnp.float32)
        z = jnp.maximum(z + b1_ref[...], 0.0)
        z = jnp.dot(z, w2_ref[...], preferred_element_type=jnp.float32)
        z = z + b2_ref[...]
        av = a_ref[0, 0]
        o_ref[...] = jnp.where(z >= 0, z, av * z)

    return pl.pallas_call(
        body,
        grid=(N // R,),
        in_specs=[
            pl.BlockSpec((R, D), lambda i: (i, 0)),
            pl.BlockSpec((_NC, R, D), lambda i: (0, i, 0)),
            pl.BlockSpec((D, D), lambda i: (0, 0)),
            pl.BlockSpec((1, D), lambda i: (0, 0)),
            pl.BlockSpec((D, D), lambda i: (0, 0)),
            pl.BlockSpec((1, D), lambda i: (0, 0)),
            pl.BlockSpec((1, 1), lambda i: (0, 0)),
        ],
        out_specs=pl.BlockSpec((R, D), lambda i: (i, 0)),
        out_shape=jax.ShapeDtypeStruct((N, D), jnp.float32),
    )(h, partials, W1, b1.reshape(1, D), W2, b2.reshape(1, D),
      a.reshape(1, 1))


def kernel(h, edge_index, edge_mask, snorm_n, W1, b1, W2, b2, prelu_a):
    del snorm_n
    N, D = h.shape
    E = edge_index.shape[1]
    eper = E // _NW
    chunk = 80
    n_chunks = eper // chunk
    srcr = edge_index[0].reshape(_NW, n_chunks, 1, chunk)
    dstr = edge_index[1].reshape(_NW, n_chunks, 1, chunk)
    maskr = lax.bitcast_convert_type(edge_mask, jnp.int32).reshape(
        _NW, n_chunks, 1, chunk)
    pk = jnp.concatenate([srcr, dstr, maskr], axis=2)
    zinit = jnp.zeros((N // _NS // 8 * 8, D), jnp.float32)
    partials = _sc_aggregate(h, pk, zinit, n_chunks, chunk)
    return _tc_mlp(h, partials, W1, b1, W2, b2, prelu_a)


# speedup vs baseline: 1.2817x; 1.1463x over previous
import functools

import jax
import jax.numpy as jnp
from jax import lax
from jax.experimental import pallas as pl
from jax.experimental.pallas import tpu as pltpu
from jax.experimental.pallas import tpu_sc as plsc

_NC = 2
_NS = 16
_NW = _NC * _NS
_L = 16


def _sc_aggregate(h, pk, zinit, n_chunks, chunk):
    N, D = h.shape
    rpt = (N // _NS) // 8 * 8
    tail = N - rpt * _NS

    mesh = plsc.VectorSubcoreMesh(core_axis_name="c", subcore_axis_name="s",
                                  num_cores=_NC, num_subcores=_NS)

    def body(h_hbm, pk_hbm, zin_hbm, out_hbm, acc_sh,
             pkt0, pkt1, pkt2, pkt3, pkt4, pkt5,
             rows0, rows1, rows2,
             semg0, semg1, semg2, sems0, sems1, sems2,
             semi0, semi1, semi2, semi3, semi4, semi5):
        cid = lax.axis_index("c")
        sid = lax.axis_index("s")
        wid = cid * _NS + sid
        pkts = (pkt0, pkt1, pkt2, pkt3, pkt4, pkt5)
        rows = (rows0, rows1, rows2)
        semg = (semg0, semg1, semg2)
        sems = (sems0, sems1, sems2)
        semi = (semi0, semi1, semi2, semi3, semi4, semi5)

        pltpu.sync_copy(zin_hbm.at[pl.ds(0, rpt)], acc_sh.at[pl.ds(sid * rpt, rpt)])
        @pl.when(sid == 0)
        def _init_tail():
            pltpu.sync_copy(zin_hbm.at[pl.ds(0, tail)],
                            acc_sh.at[pl.ds(rpt * _NS, tail)])
        plsc.subcore_barrier()

        def issue_idx(q, bi):
            pltpu.async_copy(pk_hbm.at[wid, q], pkts[bi], semi[bi])

        def wait_idx(bi):
            pltpu.make_async_copy(pk_hbm.at[wid, 0], pkts[bi], semi[bi]).wait()

        def issue_gather(br, bi):
            pltpu.async_copy(h_hbm.at[pkts[bi].at[0]], rows[br], semg[br])

        def wait_gather(br, bi):
            pltpu.make_async_copy(h_hbm.at[pkts[bi].at[0]], rows[br],
                                  semg[br]).wait()

        def issue_scatter(br, bi):
            pltpu.async_copy(rows[br], acc_sh.at[pkts[bi].at[1]], sems[br],
                             add=True)

        def wait_scatter(br, bi):
            pltpu.make_async_copy(rows[br], acc_sh.at[pkts[bi].at[1]],
                                  sems[br]).wait()

        def multiply(br, bi):
            rv, pkv = rows[br], pkts[bi]
            two = jnp.full((_L,), 2, jnp.int32)

            @plsc.parallel_loop(0, chunk, unroll=10)
            def _mul(e):
                mvi = plsc.load_gather(pkv, [two, jnp.full((_L,), e, jnp.int32)])
                mv = plsc.bitcast(mvi, jnp.float32)
                for j in range(D // _L):
                    sl = (e, pl.ds(j * _L, _L))
                    rv[sl] = rv[sl] * mv

        def phase(q, t):
            br = t % 3
            br1 = (t + 1) % 3
            bi = t % 6
            bi1 = (t + 1) % 6
            @pl.when(q >= 2)
            def _():
                wait_scatter(br1, (t + 4) % 6)
            @pl.when(q + 1 < n_chunks)
            def _():
                wait_idx(bi1)
                issue_gather(br1, bi1)
            wait_gather(br, bi)
            multiply(br, bi)
            issue_scatter(br, bi)
            @pl.when(q + 3 < n_chunks)
            def _():
                issue_idx(q + 3, (t + 3) % 6)

        issue_idx(0, 0)
        issue_idx(1, 1)
        issue_idx(2, 2)
        wait_idx(0)
        issue_gather(0, 0)

        def loop_body(s, carry):
            q0 = s * 6
            for t in range(6):
                @pl.when(q0 + t < n_chunks)
                def _(t=t):
                    phase(q0 + t, t)
            return carry

        lax.fori_loop(0, (n_chunks + 5) // 6, loop_body, 0)
        wait_scatter((n_chunks - 2) % 3, (n_chunks - 2) % 6)
        wait_scatter((n_chunks - 1) % 3, (n_chunks - 1) % 6)
        plsc.subcore_barrier()
        pltpu.sync_copy(acc_sh.at[pl.ds(sid * rpt, rpt)],
                        out_hbm.at[cid, pl.ds(sid * rpt, rpt)])
        @pl.when(sid == 0)
        def _drain_tail():
            pltpu.sync_copy(acc_sh.at[pl.ds(rpt * _NS, tail)],
                            out_hbm.at[cid, pl.ds(rpt * _NS, tail)])

    run = pl.kernel(
        body,
        out_type=jax.ShapeDtypeStruct((_NC, N, D), jnp.float32),
        mesh=mesh,
        scratch_types=[
            pltpu.VMEM_SHARED((N, D), jnp.float32),
        ] + [pltpu.VMEM((3, chunk), jnp.int32)] * 6
          + [pltpu.VMEM((chunk, D), jnp.float32)] * 3
          + [pltpu.SemaphoreType.DMA] * 12,
        compiler_params=pltpu.CompilerParams(needs_layout_passes=False),
    )
    return run(h, pk, zinit)


def _tc_mlp(h, partials, W1, b1, W2, b2, a):
    N, D = h.shape
    R = 1000

    def body(h_ref, p_ref, w1_ref, b1_ref, w2_ref, b2_ref, a_ref, o_ref):
        h2 = h_ref[...] + p_ref[0] + p_ref[1]
        z = jnp.dot(h2, w1_ref[...], preferred_element_type=jnp.float32)
        z = jnp.maximum(z + b1_ref[...], 0.0)
        z = jnp.dot(z, w2_ref[...], preferred_element_type=jnp.float32)
        z = z + b2_ref[...]
        av = a_ref[0, 0]
        o_ref[...] = jnp.where(z >= 0, z, av * z)

    return pl.pallas_call(
        body,
        grid=(N // R,),
        in_specs=[
            pl.BlockSpec((R, D), lambda i: (i, 0)),
            pl.BlockSpec((_NC, R, D), lambda i: (0, i, 0)),
            pl.BlockSpec((D, D), lambda i: (0, 0)),
            pl.BlockSpec((1, D), lambda i: (0, 0)),
            pl.BlockSpec((D, D), lambda i: (0, 0)),
            pl.BlockSpec((1, D), lambda i: (0, 0)),
            pl.BlockSpec((1, 1), lambda i: (0, 0)),
        ],
        out_specs=pl.BlockSpec((R, D), lambda i: (i, 0)),
        out_shape=jax.ShapeDtypeStruct((N, D), jnp.float32),
    )(h, partials, W1, b1.reshape(1, D), W2, b2.reshape(1, D),
      a.reshape(1, 1))


def kernel(h, edge_index, edge_mask, snorm_n, W1, b1, W2, b2, prelu_a):
    del snorm_n
    N, D = h.shape
    E = edge_index.shape[1]
    eper = E // _NW
    chunk = 100
    n_chunks = eper // chunk
    srcr = edge_index[0].reshape(_NW, n_chunks, 1, chunk)
    dstr = edge_index[1].reshape(_NW, n_chunks, 1, chunk)
    maskr = lax.bitcast_convert_type(edge_mask, jnp.int32).reshape(
        _NW, n_chunks, 1, chunk)
    pk = jnp.concatenate([srcr, dstr, maskr], axis=2)
    zinit = jnp.zeros((N // _NS // 8 * 8, D), jnp.float32)
    partials = _sc_aggregate(h, pk, zinit, n_chunks, chunk)
    return _tc_mlp(h, partials, W1, b1, W2, b2, prelu_a)


# speedup vs baseline: 1.2828x; 1.0009x over previous
import jax
import jax.numpy as jnp
from jax import lax
from jax.experimental import pallas as pl
from jax.experimental.pallas import tpu as pltpu
from jax.experimental.pallas import tpu_sc as plsc

_NC = 2
_NS = 16
_NW = _NC * _NS
_L = 16


def _sc_aggregate(h, pk, zinit, n_chunks, chunk):
    N, D = h.shape
    rpt = (N // _NS) // 8 * 8
    tail = N - rpt * _NS

    mesh = plsc.VectorSubcoreMesh(core_axis_name="c", subcore_axis_name="s",
                                  num_cores=_NC, num_subcores=_NS)

    def body(h_hbm, pk_hbm, zin_hbm, out_hbm, acc_sh,
             pkt0, pkt1, pkt2, pkt3, pkt4, pkt5,
             rows0, rows1, rows2,
             semg0, semg1, semg2, sems0, sems1, sems2,
             semi0, semi1, semi2, semi3, semi4, semi5):
        cid = lax.axis_index("c")
        sid = lax.axis_index("s")
        wid = cid * _NS + sid
        pkts = (pkt0, pkt1, pkt2, pkt3, pkt4, pkt5)
        rows = (rows0, rows1, rows2)
        semg = (semg0, semg1, semg2)
        sems = (sems0, sems1, sems2)
        semi = (semi0, semi1, semi2, semi3, semi4, semi5)

        pltpu.sync_copy(zin_hbm.at[pl.ds(0, rpt)], acc_sh.at[pl.ds(sid * rpt, rpt)])
        @pl.when(sid == 0)
        def _init_tail():
            pltpu.sync_copy(zin_hbm.at[pl.ds(0, tail)],
                            acc_sh.at[pl.ds(rpt * _NS, tail)])
        plsc.subcore_barrier()

        def issue_idx(q, bi):
            pltpu.async_copy(pk_hbm.at[wid, q], pkts[bi], semi[bi])

        def wait_idx(bi):
            pltpu.make_async_copy(pk_hbm.at[wid, 0], pkts[bi], semi[bi]).wait()

        def issue_gather(br, bi):
            pltpu.async_copy(h_hbm.at[pkts[bi].at[0]], rows[br], semg[br])

        def wait_gather(br, bi):
            pltpu.make_async_copy(h_hbm.at[pkts[bi].at[0]], rows[br],
                                  semg[br]).wait()

        def issue_scatter(br, bi):
            pltpu.async_copy(rows[br], acc_sh.at[pkts[bi].at[1]], sems[br],
                             add=True)

        def wait_scatter(br, bi):
            pltpu.make_async_copy(rows[br], acc_sh.at[pkts[bi].at[1]],
                                  sems[br]).wait()

        def multiply(br, bi):
            rv, pkv = rows[br], pkts[bi]
            two = jnp.full((_L,), 2, jnp.int32)

            @plsc.parallel_loop(0, chunk, unroll=10)
            def _mul(e):
                mvi = plsc.load_gather(pkv, [two, jnp.full((_L,), e, jnp.int32)])
                mv = plsc.bitcast(mvi, jnp.float32)
                for j in range(D // _L):
                    sl = (e, pl.ds(j * _L, _L))
                    rv[sl] = rv[sl] * mv

        def phase(q, t):
            br = t % 3
            br1 = (t + 1) % 3
            bi = t % 6
            bi1 = (t + 1) % 6
            @pl.when(q >= 2)
            def _():
                wait_scatter(br1, (t + 4) % 6)
            @pl.when(q + 1 < n_chunks)
            def _():
                wait_idx(bi1)
                issue_gather(br1, bi1)
            wait_gather(br, bi)
            multiply(br, bi)
            issue_scatter(br, bi)
            @pl.when(q + 3 < n_chunks)
            def _():
                issue_idx(q + 3, (t + 3) % 6)

        issue_idx(0, 0)
        issue_idx(1, 1)
        issue_idx(2, 2)
        wait_idx(0)
        issue_gather(0, 0)

        def loop_body(s, carry):
            q0 = s * 6
            for t in range(6):
                @pl.when(q0 + t < n_chunks)
                def _(t=t):
                    phase(q0 + t, t)
            return carry

        lax.fori_loop(0, (n_chunks + 5) // 6, loop_body, 0)
        wait_scatter((n_chunks - 2) % 3, (n_chunks - 2) % 6)
        wait_scatter((n_chunks - 1) % 3, (n_chunks - 1) % 6)
        plsc.subcore_barrier()
        pltpu.sync_copy(acc_sh.at[pl.ds(sid * rpt, rpt)],
                        out_hbm.at[cid, pl.ds(sid * rpt, rpt)])
        @pl.when(sid == 0)
        def _drain_tail():
            pltpu.sync_copy(acc_sh.at[pl.ds(rpt * _NS, tail)],
                            out_hbm.at[cid, pl.ds(rpt * _NS, tail)])

    run = pl.kernel(
        body,
        out_type=jax.ShapeDtypeStruct((_NC, N, D), jnp.float32),
        mesh=mesh,
        scratch_types=[
            pltpu.VMEM_SHARED((N, D), jnp.float32),
        ] + [pltpu.VMEM((3, chunk), jnp.int32)] * 6
          + [pltpu.VMEM((chunk, D), jnp.float32)] * 3
          + [pltpu.SemaphoreType.DMA] * 12,
        compiler_params=pltpu.CompilerParams(needs_layout_passes=False),
    )
    return run(h, pk, zinit)


def _tc_mlp(h, partials, W1, b1, W2, b2, a):
    N, D = h.shape
    R = 1000

    def body(h_ref, p_ref, w1_ref, b1_ref, w2_ref, b2_ref, a_ref, o_ref):
        h2 = h_ref[...] + p_ref[0] + p_ref[1]
        z = jnp.dot(h2, w1_ref[...], preferred_element_type=jnp.float32)
        z = jnp.maximum(z + b1_ref[...], 0.0)
        z = jnp.dot(z, w2_ref[...], preferred_element_type=jnp.float32)
        z = z + b2_ref[...]
        av = a_ref[0, 0]
        o_ref[...] = jnp.where(z >= 0, z, av * z)

    return pl.pallas_call(
        body,
        grid=(N // R,),
        in_specs=[
            pl.BlockSpec((R, D), lambda i: (i, 0)),
            pl.BlockSpec((_NC, R, D), lambda i: (0, i, 0)),
            pl.BlockSpec((D, D), lambda i: (0, 0)),
            pl.BlockSpec((1, D), lambda i: (0, 0)),
            pl.BlockSpec((D, D), lambda i: (0, 0)),
            pl.BlockSpec((1, D), lambda i: (0, 0)),
            pl.BlockSpec((1, 1), lambda i: (0, 0)),
        ],
        out_specs=pl.BlockSpec((R, D), lambda i: (i, 0)),
        out_shape=jax.ShapeDtypeStruct((N, D), jnp.float32),
    )(h, partials, W1, b1.reshape(1, D), W2, b2.reshape(1, D),
      a.reshape(1, 1))


def kernel(h, edge_index, edge_mask, snorm_n, W1, b1, W2, b2, prelu_a):
    del snorm_n
    N, D = h.shape
    E = edge_index.shape[1]
    eper = E // _NW
    chunk = 100
    n_chunks = eper // chunk
    srcr = edge_index[0].reshape(_NW, n_chunks, 1, chunk)
    dstr = edge_index[1].reshape(_NW, n_chunks, 1, chunk)
    maskr = lax.bitcast_convert_type(edge_mask, jnp.int32).reshape(
        _NW, n_chunks, 1, chunk)
    pk = jnp.concatenate([srcr, dstr, maskr], axis=2)
    zinit = jnp.zeros((N // _NS // 8 * 8, D), jnp.float32)
    partials = _sc_aggregate(h, pk, zinit, n_chunks, chunk)
    return _tc_mlp(h, partials, W1, b1, W2, b2, prelu_a)
